# baseline (speedup 1.0000x reference)
import functools

import jax
import jax.numpy as jnp
from jax import lax
from jax.experimental import pallas as pl
from jax.experimental.pallas import tpu as pltpu
from jax.experimental.pallas import tpu_sc as plsc

EPS = 1e-5

_NC = 2
_NS = 16
_NW = _NC * _NS

_GROW = 128
_GPS = 4
_SLOT_ROWS = _GPS * _GROW


def _sc_gather(flat_tab, gidx3d, b, edim):
    njq = gidx3d.shape[0]
    bpw = b // _NW
    assert bpw * _NW == b and bpw % _GROW == 0
    spw = bpw // _GROW
    nchunk = njq // 4

    mesh = plsc.VectorSubcoreMesh(core_axis_name="c", subcore_axis_name="s")

    @functools.partial(
        pl.kernel,
        mesh=mesh,
        out_type=jax.ShapeDtypeStruct((nchunk, b, 128), jnp.float32),
        compiler_params=pltpu.CompilerParams(use_tc_tiling_on_sc=False),
        scratch_types=[
            pltpu.VMEM((njq, spw, _GROW), jnp.int32),
            pltpu.VMEM((2, bpw, edim), jnp.float32),
            pltpu.SemaphoreType.DMA,
            pltpu.SemaphoreType.DMA,
            pltpu.SemaphoreType.DMA,
            pltpu.SemaphoreType.DMA,
            pltpu.SemaphoreType.DMA,
        ],
    )
    def gather_k(tab_hbm, gidx_hbm, out_hbm, idx_v, rows_v, sem_i,
                 sem_g0, sem_g1, sem_o0, sem_o1):
        sem_g = [sem_g0, sem_g1]
        sem_o = [sem_o0, sem_o1]
        wid = lax.axis_index("s") * _NC + lax.axis_index("c")
        b0 = wid * bpw
        pltpu.async_copy(gidx_hbm.at[:, pl.ds(wid * spw, spw), :], idx_v,
                         sem_i).wait()

        def outer(k, carry):
            descs = []
            for s in range(2):
                jq = 2 * k + s

                @pl.when(k >= 1)
                def _wait_out():
                    pltpu.make_async_copy(
                        rows_v.at[s],
                        out_hbm.at[0, pl.ds(0, bpw), pl.ds(0, edim)],
                        sem_o[s]).wait()

                slot_descs = []
                for t in range(spw):
                    slot_descs.append(pltpu.async_copy(
                        tab_hbm.at[idx_v.at[jq, t]],
                        rows_v.at[s, pl.ds(t * _GROW, _GROW)],
                        sem_g[s]))
                descs.append(slot_descs)
            for s in range(2):
                jq = 2 * k + s
                for d in descs[s]:
                    d.wait()
                j = lax.div(jq, 4)
                q = lax.rem(jq, 4)
                pltpu.async_copy(
                    rows_v.at[s],
                    out_hbm.at[j, pl.ds(b0, bpw), pl.ds(q * edim, edim)],
                    sem_o[s])
            return carry

        lax.fori_loop(0, njq // 2, outer, 0)
        for s in range(2):
            pltpu.make_async_copy(rows_v.at[s],
                                  out_hbm.at[0, pl.ds(0, bpw),
                                             pl.ds(0, edim)],
                                  sem_o[s]).wait()

    return gather_k(flat_tab, gidx3d)


def _prep_body(emb_ref, out_ref):
    x = emb_ref[:, 0, 0, :]
    edim, vocab = x.shape
    q = out_ref.shape[0]
    xp = jnp.pad(x, ((0, 32 - edim), (0, 4 * q - vocab)))
    y = xp.T
    out_ref[...] = jnp.concatenate(
        [y[k * q:(k + 1) * q] for k in range(4)], axis=1)


def _prep_call(emb_tables, vpad):
    nf, vocab, edim = emb_tables.shape
    rows_per_f = vpad * 32 // 128
    emb_t = jnp.transpose(emb_tables, (2, 0, 1)).reshape(edim, nf, 1, vocab)
    return pl.pallas_call(
        _prep_body,
        grid=(nf,),
        in_specs=[pl.BlockSpec((edim, 1, 1, vocab), lambda i: (0, i, 0, 0))],
        out_specs=pl.BlockSpec((rows_per_f, 128), lambda i: (i, 0)),
        out_shape=jax.ShapeDtypeStruct((nf * rows_per_f, 128), jnp.float32),
        compiler_params=pltpu.CompilerParams(
            dimension_semantics=("parallel",)),
    )(emb_t)


def _mlp_body(gath_ref, cont_ref, wct_ref, bc_ref, w1c_ref, w1d_ref,
              bias1_ref, w2_ref, bias2_ref, wo_ref, scal_ref, out_ref):
    a0 = scal_ref[0, 0]
    a1 = scal_ref[0, 1]
    a2 = scal_ref[0, 2]
    bo = scal_ref[0, 3]
    g = jnp.concatenate([gath_ref[j] for j in range(gath_ref.shape[0])],
                        axis=1)
    g = jnp.where(g >= 0, g, a0 * g)
    c = jnp.dot(cont_ref[...], wct_ref[...],
                preferred_element_type=jnp.float32) + bc_ref[...][None, :]
    c = jnp.where(c >= 0, c, a0 * c)
    z = (jnp.dot(g, w1c_ref[...], preferred_element_type=jnp.float32)
         + jnp.dot(c, w1d_ref[...], preferred_element_type=jnp.float32)
         + bias1_ref[...][None, :])
    z = jnp.where(z >= 0, z, a1 * z)
    z2 = jnp.dot(z, w2_ref[...],
                 preferred_element_type=jnp.float32) + bias2_ref[...][None, :]
    z2 = jnp.where(z2 >= 0, z2, a2 * z2)
    out_ref[...] = jnp.dot(z2, wo_ref[...],
                           preferred_element_type=jnp.float32) + bo


def _mlp_call(gath3, cont_x, wct, b_cont, w1c, w1d, bias1, w2s, bias2,
              wot, scalars, tile_b, interpret=False):
    nchunk, b, _ = gath3.shape
    n_cont = cont_x.shape[1]
    grid = (b // tile_b,)
    return pl.pallas_call(
        _mlp_body,
        grid=grid,
        in_specs=[
            pl.BlockSpec((nchunk, tile_b, 128), lambda i: (0, i, 0)),
            pl.BlockSpec((tile_b, n_cont), lambda i: (i, 0)),
            pl.BlockSpec(wct.shape, lambda i: (0, 0)),
            pl.BlockSpec(b_cont.shape, lambda i: (0,)),
            pl.BlockSpec(w1c.shape, lambda i: (0, 0)),
            pl.BlockSpec(w1d.shape, lambda i: (0, 0)),
            pl.BlockSpec(bias1.shape, lambda i: (0,)),
            pl.BlockSpec(w2s.shape, lambda i: (0, 0)),
            pl.BlockSpec(bias2.shape, lambda i: (0,)),
            pl.BlockSpec(wot.shape, lambda i: (0, 0)),
            pl.BlockSpec(scalars.shape, lambda i: (0, 0)),
        ],
        out_specs=pl.BlockSpec((tile_b, 1), lambda i: (i, 0)),
        out_shape=jax.ShapeDtypeStruct((b, 1), jnp.float32),
        compiler_params=pltpu.CompilerParams(
            dimension_semantics=("parallel",)),
        interpret=interpret,
    )(gath3, cont_x, wct, b_cont, w1c, w1d, bias1, w2s, bias2, wot, scalars)


def kernel(cat_x, cont_x, emb_tables, W_cont, b_cont, a0, W1, b1, g1, be1,
           a1, W2, b2, g2, be2, a2, Wo, bo):
    b, nf = cat_x.shape
    vocab = emb_tables.shape[1]
    edim = emb_tables.shape[2]
    epad = 32
    vpad = (vocab + 31) // 32 * 32

    q = vpad // 4
    offs = (jnp.arange(nf, dtype=jnp.int32) * vpad)[None, :]
    cat = cat_x.astype(jnp.int32)
    perm = 4 * (cat % q) + cat // q
    ids = perm + offs
    gidx3 = ids.reshape(b, nf // 4, 4).transpose(1, 2, 0).reshape(
        nf, b // _GROW, _GROW)
    flat_tab = _prep_call(emb_tables, vpad).reshape(nf * vpad, epad)

    inv = 1.0 / jnp.sqrt(jnp.float32(1.0) + EPS)
    s1 = g1 * inv
    w1s = W1.T * s1[None, :]
    bias1 = b1 * s1 + be1
    s2 = g2 * inv
    w2s = W2.T * s2[None, :]
    bias2 = b2 * s2 + be2
    n48 = W_cont.shape[0]
    scalars = jnp.concatenate([a0, a1, a2, bo]).reshape(1, 4)
    w1c = jnp.pad(w1s[n48:].reshape(nf, edim, w1s.shape[1]),
                  ((0, 0), (0, epad - edim), (0, 0)))
    w1c = w1c.reshape(nf * epad, w1s.shape[1])

    bh = b // 2
    rows_h = bh // _GROW
    outs = []
    for h in range(2):
        gidx_h = gidx3[:, h * rows_h:(h + 1) * rows_h, :]
        gath_h = _sc_gather(flat_tab, gidx_h, bh, epad)
        outs.append(_mlp_call(gath_h, cont_x[h * bh:(h + 1) * bh],
                              W_cont.T, b_cont, w1c, w1s[:n48],
                              bias1, w2s, bias2, Wo.T, scalars, tile_b=512))
    return jnp.concatenate(outs, axis=0)


# speedup vs baseline: 1.0505x; 1.0505x over previous
import functools

import jax
import jax.numpy as jnp
from jax import lax
from jax.experimental import pallas as pl
from jax.experimental.pallas import tpu as pltpu
from jax.experimental.pallas import tpu_sc as plsc

EPS = 1e-5

_NC = 2
_NS = 16
_NW = _NC * _NS

_GROW = 128
_GPS = 4
_SLOT_ROWS = _GPS * _GROW


def _sc_gather(flat_tab, gidx3d, b, edim):
    njq = gidx3d.shape[0]
    bpw = b // _NW
    assert bpw * _NW == b and bpw % _GROW == 0
    spw = bpw // _GROW
    nchunk = njq // 4

    mesh = plsc.VectorSubcoreMesh(core_axis_name="c", subcore_axis_name="s")

    @functools.partial(
        pl.kernel,
        mesh=mesh,
        out_type=jax.ShapeDtypeStruct((nchunk, b, 128), jnp.float32),
        compiler_params=pltpu.CompilerParams(use_tc_tiling_on_sc=False),
        scratch_types=[
            pltpu.VMEM((njq, spw, _GROW), jnp.int32),
            pltpu.VMEM((2, bpw, edim), jnp.float32),
            pltpu.SemaphoreType.DMA,
            pltpu.SemaphoreType.DMA,
            pltpu.SemaphoreType.DMA,
            pltpu.SemaphoreType.DMA,
            pltpu.SemaphoreType.DMA,
        ],
    )
    def gather_k(tab_hbm, gidx_hbm, out_hbm, idx_v, rows_v, sem_i,
                 sem_g0, sem_g1, sem_o0, sem_o1):
        sem_g = [sem_g0, sem_g1]
        sem_o = [sem_o0, sem_o1]
        wid = lax.axis_index("s") * _NC + lax.axis_index("c")
        b0 = wid * bpw
        pltpu.async_copy(gidx_hbm.at[:, pl.ds(wid * spw, spw), :], idx_v,
                         sem_i).wait()

        def outer(k, carry):
            descs = []
            for s in range(2):
                jq = 2 * k + s

                @pl.when(k >= 1)
                def _wait_out():
                    pltpu.make_async_copy(
                        rows_v.at[s],
                        out_hbm.at[0, pl.ds(0, bpw), pl.ds(0, edim)],
                        sem_o[s]).wait()

                slot_descs = []
                for t in range(spw):
                    slot_descs.append(pltpu.async_copy(
                        tab_hbm.at[idx_v.at[jq, t]],
                        rows_v.at[s, pl.ds(t * _GROW, _GROW)],
                        sem_g[s]))
                descs.append(slot_descs)
            for s in range(2):
                jq = 2 * k + s
                for d in descs[s]:
                    d.wait()
                j = lax.div(jq, 4)
                q = lax.rem(jq, 4)
                pltpu.async_copy(
                    rows_v.at[s],
                    out_hbm.at[j, pl.ds(b0, bpw), pl.ds(q * edim, edim)],
                    sem_o[s])
            return carry

        lax.fori_loop(0, njq // 2, outer, 0)
        for s in range(2):
            pltpu.make_async_copy(rows_v.at[s],
                                  out_hbm.at[0, pl.ds(0, bpw),
                                             pl.ds(0, edim)],
                                  sem_o[s]).wait()

    return gather_k(flat_tab, gidx3d)


def _prep_body(emb_ref, out_ref):
    x = emb_ref[:, 0, 0, :]
    edim, vocab = x.shape
    q = out_ref.shape[0]
    xp = jnp.pad(x, ((0, 32 - edim), (0, 4 * q - vocab)))
    y = xp.T
    out_ref[...] = jnp.concatenate(
        [y[k * q:(k + 1) * q] for k in range(4)], axis=1)


def _prep_call(emb_tables, vpad):
    nf, vocab, edim = emb_tables.shape
    rows_per_f = vpad * 32 // 128
    emb_t = jnp.transpose(emb_tables, (2, 0, 1)).reshape(edim, nf, 1, vocab)
    return pl.pallas_call(
        _prep_body,
        grid=(nf,),
        in_specs=[pl.BlockSpec((edim, 1, 1, vocab), lambda i: (0, i, 0, 0))],
        out_specs=pl.BlockSpec((rows_per_f, 128), lambda i: (i, 0)),
        out_shape=jax.ShapeDtypeStruct((nf * rows_per_f, 128), jnp.float32),
        compiler_params=pltpu.CompilerParams(
            dimension_semantics=("parallel",)),
    )(emb_t)


def _mlp_body(gath_ref, cont_ref, wct_ref, bc_ref, w1c_ref, w1d_ref,
              bias1_ref, w2_ref, bias2_ref, wo_ref, scal_ref, out_ref):
    a0 = scal_ref[0, 0]
    a1 = scal_ref[0, 1]
    a2 = scal_ref[0, 2]
    bo = scal_ref[0, 3]
    g = jnp.concatenate([gath_ref[j] for j in range(gath_ref.shape[0])],
                        axis=1)
    g = jnp.where(g >= 0, g, a0 * g)
    c = jnp.dot(cont_ref[...], wct_ref[...],
                preferred_element_type=jnp.float32) + bc_ref[...][None, :]
    c = jnp.where(c >= 0, c, a0 * c)
    z = (jnp.dot(g, w1c_ref[...], preferred_element_type=jnp.float32)
         + jnp.dot(c, w1d_ref[...], preferred_element_type=jnp.float32)
         + bias1_ref[...][None, :])
    z = jnp.where(z >= 0, z, a1 * z)
    z2 = jnp.dot(z, w2_ref[...],
                 preferred_element_type=jnp.float32) + bias2_ref[...][None, :]
    z2 = jnp.where(z2 >= 0, z2, a2 * z2)
    out_ref[...] = jnp.dot(z2, wo_ref[...],
                           preferred_element_type=jnp.float32) + bo


def _mlp_call(gath3, cont_x, wct, b_cont, w1c, w1d, bias1, w2s, bias2,
              wot, scalars, tile_b, interpret=False):
    nchunk, b, _ = gath3.shape
    n_cont = cont_x.shape[1]
    grid = (b // tile_b,)
    return pl.pallas_call(
        _mlp_body,
        grid=grid,
        in_specs=[
            pl.BlockSpec((nchunk, tile_b, 128), lambda i: (0, i, 0)),
            pl.BlockSpec((tile_b, n_cont), lambda i: (i, 0)),
            pl.BlockSpec(wct.shape, lambda i: (0, 0)),
            pl.BlockSpec(b_cont.shape, lambda i: (0,)),
            pl.BlockSpec(w1c.shape, lambda i: (0, 0)),
            pl.BlockSpec(w1d.shape, lambda i: (0, 0)),
            pl.BlockSpec(bias1.shape, lambda i: (0,)),
            pl.BlockSpec(w2s.shape, lambda i: (0, 0)),
            pl.BlockSpec(bias2.shape, lambda i: (0,)),
            pl.BlockSpec(wot.shape, lambda i: (0, 0)),
            pl.BlockSpec(scalars.shape, lambda i: (0, 0)),
        ],
        out_specs=pl.BlockSpec((tile_b, 1), lambda i: (i, 0)),
        out_shape=jax.ShapeDtypeStruct((b, 1), jnp.float32),
        compiler_params=pltpu.CompilerParams(
            dimension_semantics=("parallel",)),
        interpret=interpret,
    )(gath3, cont_x, wct, b_cont, w1c, w1d, bias1, w2s, bias2, wot, scalars)


def kernel(cat_x, cont_x, emb_tables, W_cont, b_cont, a0, W1, b1, g1, be1,
           a1, W2, b2, g2, be2, a2, Wo, bo):
    b, nf = cat_x.shape
    vocab = emb_tables.shape[1]
    edim = emb_tables.shape[2]
    epad = 32
    vpad = (vocab + 31) // 32 * 32

    q = vpad // 4
    offs = (jnp.arange(nf, dtype=jnp.int32) * vpad)[None, :]
    cat = cat_x.astype(jnp.int32)
    perm = 4 * (cat % q) + cat // q
    ids = perm + offs
    gidx3 = ids.reshape(b, nf // 4, 4).transpose(1, 2, 0).reshape(
        nf, b // _GROW, _GROW)
    flat_tab = _prep_call(emb_tables, vpad).reshape(nf * vpad, epad)

    gath3 = _sc_gather(flat_tab, gidx3, b, epad)

    inv = 1.0 / jnp.sqrt(jnp.float32(1.0) + EPS)
    s1 = g1 * inv
    w1s = W1.T * s1[None, :]
    bias1 = b1 * s1 + be1
    s2 = g2 * inv
    w2s = W2.T * s2[None, :]
    bias2 = b2 * s2 + be2
    n48 = W_cont.shape[0]
    scalars = jnp.concatenate([a0, a1, a2, bo]).reshape(1, 4)
    w1c = jnp.pad(w1s[n48:].reshape(nf, edim, w1s.shape[1]),
                  ((0, 0), (0, epad - edim), (0, 0)))
    w1c = w1c.reshape(nf * epad, w1s.shape[1])

    return _mlp_call(gath3, cont_x, W_cont.T, b_cont,
                     w1c, w1s[:n48],
                     bias1, w2s, bias2, Wo.T, scalars, tile_b=512)
